# baseline (speedup 1.0000x reference)
import jax
import jax.numpy as jnp
from jax.experimental import pallas as pl
from jax.experimental.pallas import tpu as pltpu

NUM_DRUGS = 1000
EMB_DIM = 128
BATCH = 1024
BB = 8
DC = 200


def _decode_kernel(ids_ref, emb_ref, bias_ref, cell_ref, out_ref, d_scratch):
    @pl.when(pl.program_id(0) == 0)
    def _gather():
        ids = ids_ref[0, :]
        onehot = (ids[:, None] == jax.lax.broadcasted_iota(
            jnp.int32, (NUM_DRUGS, NUM_DRUGS), 1)).astype(jnp.float32)
        d_scratch[...] = jax.lax.dot(
            onehot, emb_ref[...], preferred_element_type=jnp.float32)

    for j in range(NUM_DRUGS // DC):
        dd = d_scratch[j * DC:(j + 1) * DC, :]
        cols = []
        for b in range(BB):
            p = dd * cell_ref[b, j * DC:(j + 1) * DC, :]
            cols.append(jnp.sum(p, axis=1, keepdims=True))
        red_t = jnp.concatenate(cols, axis=1)
        out_ref[0, j * DC:(j + 1) * DC, :] = (
            red_t + bias_ref[j * DC:(j + 1) * DC, :])


def kernel(cell_repr, drg_ids, emb_table, drg_bias):
    ids2d = drg_ids.astype(jnp.int32).reshape(1, NUM_DRUGS)
    bias2d = drg_bias.reshape(NUM_DRUGS, 1)
    grid = (BATCH // BB,)
    out_t = pl.pallas_call(
        _decode_kernel,
        grid=grid,
        in_specs=[
            pl.BlockSpec((1, NUM_DRUGS), lambda i: (0, 0)),
            pl.BlockSpec((NUM_DRUGS, EMB_DIM), lambda i: (0, 0)),
            pl.BlockSpec((NUM_DRUGS, 1), lambda i: (0, 0)),
            pl.BlockSpec((BB, NUM_DRUGS, EMB_DIM), lambda i: (i, 0, 0)),
        ],
        out_specs=pl.BlockSpec((1, NUM_DRUGS, BB), lambda i: (i, 0, 0)),
        out_shape=jax.ShapeDtypeStruct((BATCH // BB, NUM_DRUGS, BB), jnp.float32),
        scratch_shapes=[pltpu.VMEM((NUM_DRUGS, EMB_DIM), jnp.float32)],
    )(ids2d, emb_table, bias2d, cell_repr)
    return out_t.transpose(0, 2, 1).reshape(BATCH, NUM_DRUGS)


# speedup vs baseline: 1.5187x; 1.5187x over previous
import jax
import jax.numpy as jnp
from jax.experimental import pallas as pl
from jax.experimental.pallas import tpu as pltpu

NUM_DRUGS = 1000
EMB_DIM = 128
BATCH = 1024
BB = 32
DC = 200


def _decode_kernel(ids_ref, emb_ref, bias_ref, cell_ref, out_ref, d_scratch):
    @pl.when(pl.program_id(0) == 0)
    def _gather():
        ids = ids_ref[0, :]
        onehot = (ids[:, None] == jax.lax.broadcasted_iota(
            jnp.int32, (NUM_DRUGS, NUM_DRUGS), 1)).astype(jnp.float32)
        d_scratch[...] = jax.lax.dot(
            onehot, emb_ref[...], preferred_element_type=jnp.float32)

    for j in range(NUM_DRUGS // DC):
        dd = d_scratch[j * DC:(j + 1) * DC, :]
        cols = []
        for b in range(BB):
            p = dd * cell_ref[b, j * DC:(j + 1) * DC, :]
            cols.append(jnp.sum(p, axis=1, keepdims=True))
        red_t = jnp.concatenate(cols, axis=1)
        out_ref[0, j * DC:(j + 1) * DC, :] = (
            red_t + bias_ref[j * DC:(j + 1) * DC, :])


def kernel(cell_repr, drg_ids, emb_table, drg_bias):
    ids2d = drg_ids.astype(jnp.int32).reshape(1, NUM_DRUGS)
    bias2d = drg_bias.reshape(NUM_DRUGS, 1)
    grid = (BATCH // BB,)
    out_t = pl.pallas_call(
        _decode_kernel,
        grid=grid,
        in_specs=[
            pl.BlockSpec((1, NUM_DRUGS), lambda i: (0, 0)),
            pl.BlockSpec((NUM_DRUGS, EMB_DIM), lambda i: (0, 0)),
            pl.BlockSpec((NUM_DRUGS, 1), lambda i: (0, 0)),
            pl.BlockSpec((BB, NUM_DRUGS, EMB_DIM), lambda i: (i, 0, 0)),
        ],
        out_specs=pl.BlockSpec((1, NUM_DRUGS, BB), lambda i: (i, 0, 0)),
        out_shape=jax.ShapeDtypeStruct((BATCH // BB, NUM_DRUGS, BB), jnp.float32),
        scratch_shapes=[pltpu.VMEM((NUM_DRUGS, EMB_DIM), jnp.float32)],
    )(ids2d, emb_table, bias2d, cell_repr)
    return out_t.transpose(0, 2, 1).reshape(BATCH, NUM_DRUGS)
